# baseline (speedup 1.0000x reference)
import jax
import jax.numpy as jnp
from jax.experimental import pallas as pl
from jax.experimental.pallas import tpu as pltpu

_BB = 128


def _crf_block_kernel(featsT_ref, transB_ref, transT_ref, score_ref, path_ref,
                      hist_ref):
    T, S, BB = featsT_ref.shape
    sidx = jax.lax.broadcasted_iota(jnp.int32, (S, BB), 0).astype(jnp.float32)

    delta0 = jnp.full((S, BB), -10000.0, dtype=jnp.float32)
    hist_ref[0] = delta0

    def fwd(t, delta):
        feat_t = featsT_ref[t]
        scores = transB_ref[...] + delta[None, :, :]
        m = jnp.max(scores, axis=1)
        new_delta = m + feat_t
        hist_ref[t] = new_delta
        return new_delta

    final_delta = jax.lax.fori_loop(1, T, fwd, delta0, unroll=2)

    m2 = jnp.max(final_delta, axis=0, keepdims=True)
    score_ref[0] = m2
    last_tag = jnp.min(
        jnp.where(final_delta == m2, sidx, float(S)), axis=0, keepdims=True
    )
    path_ref[T - 1] = last_tag.astype(jnp.int32)

    transT = transT_ref[...]

    def bwd(k, tag):
        t = T - 1 - k
        onehot = jnp.where(sidx == tag, 1.0, 0.0)
        trans_row = jax.lax.dot_general(
            transT, onehot,
            dimension_numbers=(((1,), (0,)), ((), ())),
            preferred_element_type=jnp.float32,
            precision=jax.lax.Precision.HIGHEST,
        )
        cand = trans_row + hist_ref[t - 1]
        mb = jnp.max(cand, axis=0, keepdims=True)
        cur = jnp.min(
            jnp.where(cand == mb, sidx, float(S)), axis=0, keepdims=True
        )
        path_ref[t - 1] = cur.astype(jnp.int32)
        return cur

    jax.lax.fori_loop(0, T - 1, bwd, last_tag, unroll=2)


def kernel(feats, transitions):
    B, T, S = feats.shape
    bb = _BB
    grid = (B // bb,)

    featsT = jnp.transpose(feats, (1, 2, 0))
    transB = jnp.broadcast_to(transitions[:, :, None], (S, S, bb))
    transT = transitions.T

    score, pathT = pl.pallas_call(
        _crf_block_kernel,
        grid=grid,
        in_specs=[
            pl.BlockSpec((T, S, bb), lambda b: (0, 0, b)),
            pl.BlockSpec((S, S, bb), lambda b: (0, 0, 0)),
            pl.BlockSpec((S, S), lambda b: (0, 0)),
        ],
        out_specs=[
            pl.BlockSpec((1, 1, bb), lambda b: (0, 0, b)),
            pl.BlockSpec((T, 1, bb), lambda b: (0, 0, b)),
        ],
        out_shape=[
            jax.ShapeDtypeStruct((1, 1, B), jnp.float32),
            jax.ShapeDtypeStruct((T, 1, B), jnp.int32),
        ],
        scratch_shapes=[pltpu.VMEM((T, S, bb), jnp.float32)],
        compiler_params=pltpu.CompilerParams(
            dimension_semantics=("arbitrary",),
        ),
    )(featsT, transB, transT)

    return score.reshape(B), pathT.reshape(T, B).T


# speedup vs baseline: 1.2605x; 1.2605x over previous
import jax
import jax.numpy as jnp
from jax.experimental import pallas as pl
from jax.experimental.pallas import tpu as pltpu

_LANES = 128
_BB = 256


def _crf_block_kernel(featsT_ref, transB_ref, transT_ref, score_ref, path_ref,
                      hist_ref):
    T, S, BB = featsT_ref.shape
    L = transB_ref.shape[2]
    sidx = jax.lax.broadcasted_iota(jnp.int32, (S, BB), 0).astype(jnp.float32)

    hist_ref[0] = jnp.full((S, BB), -10000.0, dtype=jnp.float32)

    def fwd(t, carry):
        dA, dB = carry
        feat_t = featsT_ref[t]
        tb = transB_ref[...]
        mA = jnp.max(tb + dA[None, :, :], axis=1)
        mB = jnp.max(tb + dB[None, :, :], axis=1)
        ndA = mA + feat_t[:, :L]
        ndB = mB + feat_t[:, L:]
        hist_ref[t, :, :L] = ndA
        hist_ref[t, :, L:] = ndB
        return ndA, ndB

    d0 = jnp.full((S, L), -10000.0, dtype=jnp.float32)
    fA, fB = jax.lax.fori_loop(1, T, fwd, (d0, d0), unroll=2)
    final_delta = jnp.concatenate([fA, fB], axis=1)

    m2 = jnp.max(final_delta, axis=0, keepdims=True)
    score_ref[0] = m2
    last_tag = jnp.min(
        jnp.where(final_delta == m2, sidx, float(S)), axis=0, keepdims=True
    )
    path_ref[T - 1] = last_tag.astype(jnp.int32)

    transT = transT_ref[...]

    def bwd(k, tag):
        t = T - 1 - k
        onehot = jnp.where(sidx == tag, 1.0, 0.0)
        trans_row = jax.lax.dot_general(
            transT, onehot,
            dimension_numbers=(((1,), (0,)), ((), ())),
            preferred_element_type=jnp.float32,
            precision=jax.lax.Precision.HIGHEST,
        )
        cand = trans_row + hist_ref[t - 1]
        mb = jnp.max(cand, axis=0, keepdims=True)
        cur = jnp.min(
            jnp.where(cand == mb, sidx, float(S)), axis=0, keepdims=True
        )
        path_ref[t - 1] = cur.astype(jnp.int32)
        return cur

    jax.lax.fori_loop(0, T - 1, bwd, last_tag, unroll=2)


def kernel(feats, transitions):
    B, T, S = feats.shape
    bb = _BB
    grid = (B // bb,)

    featsT = jnp.transpose(feats, (1, 2, 0))
    transB = jnp.broadcast_to(transitions[:, :, None], (S, S, _LANES))
    transT = transitions.T

    score, pathT = pl.pallas_call(
        _crf_block_kernel,
        grid=grid,
        in_specs=[
            pl.BlockSpec((T, S, bb), lambda b: (0, 0, b)),
            pl.BlockSpec((S, S, _LANES), lambda b: (0, 0, 0)),
            pl.BlockSpec((S, S), lambda b: (0, 0)),
        ],
        out_specs=[
            pl.BlockSpec((1, 1, bb), lambda b: (0, 0, b)),
            pl.BlockSpec((T, 1, bb), lambda b: (0, 0, b)),
        ],
        out_shape=[
            jax.ShapeDtypeStruct((1, 1, B), jnp.float32),
            jax.ShapeDtypeStruct((T, 1, B), jnp.int32),
        ],
        scratch_shapes=[pltpu.VMEM((T, S, bb), jnp.float32)],
        compiler_params=pltpu.CompilerParams(
            dimension_semantics=("arbitrary",),
        ),
    )(featsT, transB, transT)

    return score.reshape(B), pathT.reshape(T, B).T


# speedup vs baseline: 1.4412x; 1.1434x over previous
import jax
import jax.numpy as jnp
from jax.experimental import pallas as pl
from jax.experimental.pallas import tpu as pltpu

_LANES = 128
_BB = 256


def _crf_block_kernel(featsT_ref, transB_ref, transT_ref, score_ref, path_ref,
                      hist_ref):
    T, S, BB = featsT_ref.shape
    L = transB_ref.shape[2]
    sidx = jax.lax.broadcasted_iota(jnp.int32, (S, BB), 0).astype(jnp.float32)

    hist_ref[0] = jnp.full((S, BB), -10000.0, dtype=jnp.float32)

    def fwd(t, carry):
        dA, dB = carry
        feat_t = featsT_ref[t]
        mA = mB = None
        for h in range(2):
            aA = aB = None
            for jj in range(S // 2):
                j = h * (S // 2) + jj
                row = transB_ref[j]
                cA = row + dA[j:j + 1, :]
                cB = row + dB[j:j + 1, :]
                aA = cA if aA is None else jnp.maximum(aA, cA)
                aB = cB if aB is None else jnp.maximum(aB, cB)
            mA = aA if mA is None else jnp.maximum(mA, aA)
            mB = aB if mB is None else jnp.maximum(mB, aB)
        ndA = mA + feat_t[:, :L]
        ndB = mB + feat_t[:, L:]
        hist_ref[t, :, :L] = ndA
        hist_ref[t, :, L:] = ndB
        return ndA, ndB

    d0 = jnp.full((S, L), -10000.0, dtype=jnp.float32)
    fA, fB = jax.lax.fori_loop(1, T, fwd, (d0, d0), unroll=2)
    final_delta = jnp.concatenate([fA, fB], axis=1)

    m2 = jnp.max(final_delta, axis=0, keepdims=True)
    score_ref[0] = m2
    last_tag = jnp.min(
        jnp.where(final_delta == m2, sidx, float(S)), axis=0, keepdims=True
    )
    path_ref[T - 1] = last_tag.astype(jnp.int32)

    transT = transT_ref[...]

    def bwd(k, tag):
        t = T - 1 - k
        onehot = jnp.where(sidx == tag, 1.0, 0.0)
        trans_row = jax.lax.dot_general(
            transT, onehot,
            dimension_numbers=(((1,), (0,)), ((), ())),
            preferred_element_type=jnp.float32,
            precision=jax.lax.Precision.HIGHEST,
        )
        cand = trans_row + hist_ref[t - 1]
        mb = jnp.max(cand, axis=0, keepdims=True)
        cur = jnp.min(
            jnp.where(cand == mb, sidx, float(S)), axis=0, keepdims=True
        )
        path_ref[t - 1] = cur.astype(jnp.int32)
        return cur

    jax.lax.fori_loop(0, T - 1, bwd, last_tag, unroll=2)


def kernel(feats, transitions):
    B, T, S = feats.shape
    bb = _BB
    grid = (B // bb,)

    featsT = jnp.transpose(feats, (1, 2, 0))
    transB = jnp.broadcast_to(transitions.T[:, :, None], (S, S, _LANES))
    transT = transitions.T

    score, pathT = pl.pallas_call(
        _crf_block_kernel,
        grid=grid,
        in_specs=[
            pl.BlockSpec((T, S, bb), lambda b: (0, 0, b)),
            pl.BlockSpec((S, S, _LANES), lambda b: (0, 0, 0)),
            pl.BlockSpec((S, S), lambda b: (0, 0)),
        ],
        out_specs=[
            pl.BlockSpec((1, 1, bb), lambda b: (0, 0, b)),
            pl.BlockSpec((T, 1, bb), lambda b: (0, 0, b)),
        ],
        out_shape=[
            jax.ShapeDtypeStruct((1, 1, B), jnp.float32),
            jax.ShapeDtypeStruct((T, 1, B), jnp.int32),
        ],
        scratch_shapes=[pltpu.VMEM((T, S, bb), jnp.float32)],
        compiler_params=pltpu.CompilerParams(
            dimension_semantics=("arbitrary",),
        ),
    )(featsT, transB, transT)

    return score.reshape(B), pathT.reshape(T, B).T
